# baseline (speedup 1.0000x reference)
import functools

import jax
import jax.numpy as jnp
from jax import lax
from jax.experimental import pallas as pl
from jax.experimental.pallas import tpu as pltpu
from jax.experimental.pallas import tpu_sc as plsc

N_NUM = 100
N_CAT = 26
VOCAB = 1000
EMBED_DIM = 128
BATCH = 16384

NUM_CORES = 2
NUM_SUBCORES = 16
NW = NUM_CORES * NUM_SUBCORES

GB = 4
GROUP = GB * N_CAT
NGRP = BATCH // GB
K = 4
CHB = BATCH // K
CHGRP = NGRP // K
GRP_PER_W = CHGRP // NW
NBUF = 4
CHUNKS = (0, 16, 32, 48, 64, 80, 88)

RET_BM = 256
RET_STEPS = CHB // RET_BM


def _sc_gather_chunk(tables_flat, idx2d, kc):
    mesh = plsc.VectorSubcoreMesh(core_axis_name="c", subcore_axis_name="s")

    @functools.partial(
        pl.kernel,
        mesh=mesh,
        out_type=jax.ShapeDtypeStruct((CHB * N_CAT, EMBED_DIM), jnp.float32),
        scratch_types=[
            pltpu.VMEM((GRP_PER_W, GROUP), jnp.int32),
            pltpu.VMEM((GRP_PER_W, GROUP), jnp.int32),
            pltpu.VMEM((GROUP,), jnp.int32),
            pltpu.VMEM((NBUF, GROUP, EMBED_DIM), jnp.float32),
            [pltpu.SemaphoreType.DMA] * NBUF,
            [pltpu.SemaphoreType.DMA] * NBUF,
        ],
    )
    def k(tab_hbm, idx_hbm, out_hbm, idx_v, flat_v, off_v, bufs,
          gsems, osems):
        wid = lax.axis_index("s") * NUM_CORES + lax.axis_index("c")
        gbase = kc * CHGRP + wid * GRP_PER_W
        lbase = wid * GRP_PER_W
        pltpu.sync_copy(idx_hbm.at[pl.ds(gbase, GRP_PER_W)], idx_v)
        lane = lax.iota(jnp.int32, 16)
        for o in CHUNKS:
            off_v[pl.ds(o, 16)] = lax.rem(o + lane, N_CAT) * VOCAB

        def flats(g):
            for o in CHUNKS:
                flat_v[g, pl.ds(o, 16)] = (
                    off_v[pl.ds(o, 16)] + idx_v[g, pl.ds(o, 16)]
                )

        def gather(g, b):
            return pltpu.make_async_copy(
                tab_hbm.at[flat_v.at[g]], bufs.at[b], gsems[b])

        def out_copy(g, b):
            return pltpu.make_async_copy(
                bufs.at[b],
                out_hbm.at[pl.ds((lbase + g) * GROUP, GROUP)], osems[b])

        for b in range(NBUF):
            flats(b)
            gather(b, b).start()

        def step(go, carry):
            for b in range(NBUF):
                g = go * NBUF + b
                gather(g, b).wait()
                out_copy(g, b).start()
                gn = g + NBUF

                @pl.when(gn < GRP_PER_W)
                def _():
                    out_copy(g, b).wait()
                    flats(gn)
                    gather(gn, b).start()

            return carry

        lax.fori_loop(0, GRP_PER_W // NBUF, step, 0)
        for b in range(NBUF):
            out_copy(GRP_PER_W - NBUF + b, b).wait()

    return k(tables_flat, idx2d)


def _retile_chunk(src2d, carry, kc):
    have_carry = carry is not None

    def body(*refs):
        s_ref, o_ref = refs[0], refs[-1]
        o_ref[...] = s_ref[...].reshape(RET_BM, N_CAT, EMBED_DIM)

    in_specs = [pl.BlockSpec((RET_BM * N_CAT, EMBED_DIM), lambda i: (i, 0))]
    args = [src2d]
    if have_carry:
        in_specs.append(pl.BlockSpec(memory_space=pl.ANY))
        args.append(carry)
    return pl.pallas_call(
        body,
        grid=(RET_STEPS,),
        in_specs=in_specs,
        out_specs=pl.BlockSpec(
            (RET_BM, N_CAT, EMBED_DIM),
            lambda i, _k=kc: (_k * RET_STEPS + i, 0, 0)),
        out_shape=jax.ShapeDtypeStruct((BATCH, N_CAT, EMBED_DIM),
                                       jnp.float32),
        input_output_aliases={1: 0} if have_carry else {},
    )(*args)


def _mlp(x_num, W1, b1, W2, b2):
    BM = 1024

    def body(x_ref, w1_ref, b1_ref, w2_ref, b2_ref, o_ref):
        h = jnp.dot(x_ref[...], w1_ref[...],
                    preferred_element_type=jnp.float32) + b1_ref[...]
        h = jnp.maximum(h, 0.0)
        o_ref[...] = jnp.dot(h, w2_ref[...],
                             preferred_element_type=jnp.float32) + b2_ref[...]

    return pl.pallas_call(
        body,
        grid=(BATCH // BM,),
        in_specs=[
            pl.BlockSpec((BM, N_NUM), lambda i: (i, 0)),
            pl.BlockSpec((N_NUM, EMBED_DIM), lambda i: (0, 0)),
            pl.BlockSpec((1, EMBED_DIM), lambda i: (0, 0)),
            pl.BlockSpec((EMBED_DIM, EMBED_DIM), lambda i: (0, 0)),
            pl.BlockSpec((1, EMBED_DIM), lambda i: (0, 0)),
        ],
        out_specs=pl.BlockSpec((BM, EMBED_DIM), lambda i: (i, 0)),
        out_shape=jax.ShapeDtypeStruct((BATCH, EMBED_DIM), jnp.float32),
    )(x_num, W1, b1.reshape(1, EMBED_DIM), W2, b2.reshape(1, EMBED_DIM))


def kernel(x_num, x_cat, W1, b1, W2, b2, tables):
    idx2d = x_cat.astype(jnp.int32).reshape(NGRP, GROUP)
    tables_flat = tables.reshape(N_CAT * VOCAB, EMBED_DIM)
    chunks = [_sc_gather_chunk(tables_flat, idx2d, kc) for kc in range(K)]
    x_cats = _retile_chunk(chunks[0], None, 0)
    for kc in range(1, K):
        x_cats = _retile_chunk(chunks[kc], x_cats, kc)
    num_out = _mlp(x_num, W1, b1, W2, b2)[:, None, :]
    return (num_out, x_cats)


# speedup vs baseline: 2.8975x; 2.8975x over previous
import functools

import jax
import jax.numpy as jnp
from jax import lax
from jax.experimental import pallas as pl
from jax.experimental.pallas import tpu as pltpu
from jax.experimental.pallas import tpu_sc as plsc

N_NUM = 100
N_CAT = 26
VOCAB = 1000
EMBED_DIM = 128
BATCH = 16384

NUM_CORES = 2
NUM_SUBCORES = 16
NW = NUM_CORES * NUM_SUBCORES

ROWS = BATCH * N_CAT
BLK = 128
IDX_ROWS = ROWS // BLK
IDX_PER_W = IDX_ROWS // NW
FBLK = BATCH // BLK
NBUF = 4


def _sc_gather(tables_flat, idxt2d):
    mesh = plsc.VectorSubcoreMesh(core_axis_name="c", subcore_axis_name="s")

    @functools.partial(
        pl.kernel,
        mesh=mesh,
        out_type=jax.ShapeDtypeStruct((ROWS, EMBED_DIM), jnp.float32),
        scratch_types=[
            pltpu.VMEM((IDX_PER_W, BLK), jnp.int32),
            pltpu.VMEM((NBUF, BLK, EMBED_DIM), jnp.float32),
            [pltpu.SemaphoreType.DMA] * NBUF,
            [pltpu.SemaphoreType.DMA] * NBUF,
        ],
    )
    def k(tab_hbm, idx_hbm, out_hbm, idx_v, bufs, gsems, osems):
        wid = lax.axis_index("s") * NUM_CORES + lax.axis_index("c")
        base = wid * IDX_PER_W
        pltpu.sync_copy(idx_hbm.at[pl.ds(base, IDX_PER_W)], idx_v)

        def flats(j):
            field = lax.div(base + j, FBLK)
            off = field * VOCAB
            for c in range(BLK // 16):
                idx_v[j, pl.ds(c * 16, 16)] = (
                    off + idx_v[j, pl.ds(c * 16, 16)]
                )

        def gather(j, b):
            return pltpu.make_async_copy(
                tab_hbm.at[idx_v.at[j]], bufs.at[b], gsems[b])

        def out_copy(j, b):
            return pltpu.make_async_copy(
                bufs.at[b], out_hbm.at[pl.ds((base + j) * BLK, BLK)],
                osems[b])

        for b in range(NBUF):
            flats(b)
            gather(b, b).start()

        def step(jo, carry):
            for b in range(NBUF):
                j = jo * NBUF + b
                gather(j, b).wait()
                out_copy(j, b).start()
                jn = j + NBUF

                @pl.when(jn < IDX_PER_W)
                def _():
                    out_copy(j, b).wait()
                    flats(jn)
                    gather(jn, b).start()

            return carry

        lax.fori_loop(0, IDX_PER_W // NBUF, step, 0)
        for b in range(NBUF):
            out_copy(IDX_PER_W - NBUF + b, b).wait()

    return k(tables_flat, idxt2d)


def _mlp(x_num, W1, b1, W2, b2):
    BM = 1024

    def body(x_ref, w1_ref, b1_ref, w2_ref, b2_ref, o_ref):
        h = jnp.dot(x_ref[...], w1_ref[...],
                    preferred_element_type=jnp.float32) + b1_ref[...]
        h = jnp.maximum(h, 0.0)
        o_ref[...] = jnp.dot(h, w2_ref[...],
                             preferred_element_type=jnp.float32) + b2_ref[...]

    return pl.pallas_call(
        body,
        grid=(BATCH // BM,),
        in_specs=[
            pl.BlockSpec((BM, N_NUM), lambda i: (i, 0)),
            pl.BlockSpec((N_NUM, EMBED_DIM), lambda i: (0, 0)),
            pl.BlockSpec((1, EMBED_DIM), lambda i: (0, 0)),
            pl.BlockSpec((EMBED_DIM, EMBED_DIM), lambda i: (0, 0)),
            pl.BlockSpec((1, EMBED_DIM), lambda i: (0, 0)),
        ],
        out_specs=pl.BlockSpec((BM, EMBED_DIM), lambda i: (i, 0)),
        out_shape=jax.ShapeDtypeStruct((BATCH, EMBED_DIM), jnp.float32),
    )(x_num, W1, b1.reshape(1, EMBED_DIM), W2, b2.reshape(1, EMBED_DIM))


def kernel(x_num, x_cat, W1, b1, W2, b2, tables):
    idxt2d = x_cat.astype(jnp.int32).T.reshape(IDX_ROWS, BLK)
    tables_flat = tables.reshape(N_CAT * VOCAB, EMBED_DIM)
    out2d = _sc_gather(tables_flat, idxt2d)
    x_cats = out2d.reshape(N_CAT, BATCH, EMBED_DIM).transpose(1, 0, 2)
    num_out = _mlp(x_num, W1, b1, W2, b2)[:, None, :]
    return (num_out, x_cats)


# speedup vs baseline: 2.9177x; 1.0070x over previous
import functools

import jax
import jax.numpy as jnp
from jax import lax
from jax.experimental import pallas as pl
from jax.experimental.pallas import tpu as pltpu
from jax.experimental.pallas import tpu_sc as plsc

N_NUM = 100
N_CAT = 26
VOCAB = 1000
EMBED_DIM = 128
BATCH = 16384

NUM_CORES = 2
NUM_SUBCORES = 16
NW = NUM_CORES * NUM_SUBCORES

ROWS = BATCH * N_CAT
BLK = 128
IDX_ROWS = ROWS // BLK
IDX_PER_W = IDX_ROWS // NW
FBLK = BATCH // BLK
NBUF = 6
NDEF = 2


def _sc_gather(tables_flat, idxt2d):
    mesh = plsc.VectorSubcoreMesh(core_axis_name="c", subcore_axis_name="s")

    @functools.partial(
        pl.kernel,
        mesh=mesh,
        out_type=jax.ShapeDtypeStruct((ROWS, EMBED_DIM), jnp.float32),
        scratch_types=[
            pltpu.VMEM((IDX_PER_W, BLK), jnp.int32),
            pltpu.VMEM((NBUF, BLK, EMBED_DIM), jnp.float32),
            [pltpu.SemaphoreType.DMA] * NBUF,
            [pltpu.SemaphoreType.DMA] * NBUF,
        ],
    )
    def k(tab_hbm, idx_hbm, out_hbm, idx_v, bufs, gsems, osems):
        wid = lax.axis_index("s") * NUM_CORES + lax.axis_index("c")
        base = wid * IDX_PER_W
        pltpu.sync_copy(idx_hbm.at[pl.ds(base, IDX_PER_W)], idx_v)

        def flats(j):
            field = lax.div(base + j, FBLK)
            off = field * VOCAB
            for c in range(BLK // 16):
                idx_v[j, pl.ds(c * 16, 16)] = (
                    off + idx_v[j, pl.ds(c * 16, 16)]
                )

        def gather(j, b):
            return pltpu.make_async_copy(
                tab_hbm.at[idx_v.at[j]], bufs.at[b], gsems[b])

        def out_copy(j, b):
            return pltpu.make_async_copy(
                bufs.at[b], out_hbm.at[pl.ds((base + j) * BLK, BLK)],
                osems[b])

        for b in range(NBUF):
            flats(b)
            gather(b, b).start()

        def step(jo, carry):
            for b in range(NBUF):
                j = jo * NBUF + b

                @pl.when(j < IDX_PER_W)
                def _():
                    gather(j, b).wait()
                    out_copy(j, b).start()

                jd = j - NDEF
                bd = (b - NDEF) % NBUF
                jn = jd + NBUF

                @pl.when(jnp.logical_and(jd >= 0, jn < IDX_PER_W))
                def _():
                    out_copy(jd, bd).wait()
                    flats(jn)
                    gather(jn, bd).start()

            return carry

        nsteps = (IDX_PER_W + NBUF - 1) // NBUF
        lax.fori_loop(0, nsteps, step, 0)
        for t in range(NBUF):
            j = IDX_PER_W - NBUF + t
            out_copy(j, j % NBUF).wait()

    return k(tables_flat, idxt2d)


def _mlp(x_num, W1, b1, W2, b2):
    BM = 1024

    def body(x_ref, w1_ref, b1_ref, w2_ref, b2_ref, o_ref):
        h = jnp.dot(x_ref[...], w1_ref[...],
                    preferred_element_type=jnp.float32) + b1_ref[...]
        h = jnp.maximum(h, 0.0)
        o_ref[...] = jnp.dot(h, w2_ref[...],
                             preferred_element_type=jnp.float32) + b2_ref[...]

    return pl.pallas_call(
        body,
        grid=(BATCH // BM,),
        in_specs=[
            pl.BlockSpec((BM, N_NUM), lambda i: (i, 0)),
            pl.BlockSpec((N_NUM, EMBED_DIM), lambda i: (0, 0)),
            pl.BlockSpec((1, EMBED_DIM), lambda i: (0, 0)),
            pl.BlockSpec((EMBED_DIM, EMBED_DIM), lambda i: (0, 0)),
            pl.BlockSpec((1, EMBED_DIM), lambda i: (0, 0)),
        ],
        out_specs=pl.BlockSpec((BM, EMBED_DIM), lambda i: (i, 0)),
        out_shape=jax.ShapeDtypeStruct((BATCH, EMBED_DIM), jnp.float32),
    )(x_num, W1, b1.reshape(1, EMBED_DIM), W2, b2.reshape(1, EMBED_DIM))


def kernel(x_num, x_cat, W1, b1, W2, b2, tables):
    idxt2d = x_cat.astype(jnp.int32).T.reshape(IDX_ROWS, BLK)
    tables_flat = tables.reshape(N_CAT * VOCAB, EMBED_DIM)
    out2d = _sc_gather(tables_flat, idxt2d)
    x_cats = out2d.reshape(N_CAT, BATCH, EMBED_DIM).transpose(1, 0, 2)
    num_out = _mlp(x_num, W1, b1, W2, b2)[:, None, :]
    return (num_out, x_cats)
